# Initial kernel scaffold; baseline (speedup 1.0000x reference)
import jax
import jax.numpy as jnp
from jax.experimental import pallas as pl


def kernel(x, pos_table):
    raise NotImplementedError("write your pallas kernel here")


# speedup vs baseline: 1.4298x; 1.4298x over previous
import jax
import jax.numpy as jnp
from jax.experimental import pallas as pl


def kernel(x, pos_table):
    B, S, D = x.shape
    BLK = 512

    def body(x_ref, p_ref, o_ref):
        o_ref[...] = x_ref[...] + p_ref[...][None]

    return pl.pallas_call(
        body,
        grid=(S // BLK, B),
        in_specs=[
            pl.BlockSpec((1, BLK, D), lambda i, b: (b, i, 0)),
            pl.BlockSpec((BLK, D), lambda i, b: (i, 0)),
        ],
        out_specs=pl.BlockSpec((1, BLK, D), lambda i, b: (b, i, 0)),
        out_shape=jax.ShapeDtypeStruct((B, S, D), x.dtype),
    )(x, pos_table)
